# baseline (device time: 79247 ns/iter reference)
import jax
import jax.numpy as jnp
from jax import lax
from jax.experimental import pallas as pl
from jax.experimental.pallas import tpu as pltpu

N_DEV = 4
N_LAYERS = 3
N_HOPS = N_DEV - 1


def kernel(x, Win0, Wout0, Win1, Wout1, Win2, Wout2):
    b, d = x.shape

    def body(x_ref, win0, wout0, win1, wout1, win2, wout2, out_ref,
             comm_ref, send_sems, recv_sems):
        my = lax.axis_index("i")
        left = (my - 1) % N_DEV
        right = (my + 1) % N_DEV

        barrier = pltpu.get_barrier_semaphore()
        for nbr in (left, right):
            pl.semaphore_signal(
                barrier, inc=1,
                device_id=(nbr,), device_id_type=pl.DeviceIdType.MESH,
            )
        pl.semaphore_wait(barrier, 2)

        xv = x_ref[:, :]
        layers = [(win0, wout0), (win1, wout1), (win2, wout2)]
        for l, (win, wout) in enumerate(layers):
            h = jnp.maximum(
                jnp.dot(xv, win[:, :], preferred_element_type=jnp.float32), 0.0)
            partial = jnp.dot(h, wout[:, :], preferred_element_type=jnp.float32)

            base = l * N_DEV
            comm_ref[base, :, :] = partial
            acc = partial
            for hop in range(N_HOPS):
                sem = l * N_HOPS + hop
                rdma = pltpu.make_async_remote_copy(
                    src_ref=comm_ref.at[base + hop],
                    dst_ref=comm_ref.at[base + hop + 1],
                    send_sem=send_sems.at[sem],
                    recv_sem=recv_sems.at[sem],
                    device_id=(right,),
                    device_id_type=pl.DeviceIdType.MESH,
                )
                rdma.start()
                rdma.wait()
                acc = acc + comm_ref[base + hop + 1, :, :]
            xv = acc
        out_ref[:, :] = xv

    return pl.pallas_call(
        body,
        out_shape=jax.ShapeDtypeStruct((b, d), jnp.float32),
        in_specs=[pl.BlockSpec(memory_space=pltpu.VMEM)] * 7,
        out_specs=pl.BlockSpec(memory_space=pltpu.VMEM),
        scratch_shapes=[
            pltpu.VMEM((N_LAYERS * N_DEV, b, d), jnp.float32),
            pltpu.SemaphoreType.DMA((N_LAYERS * N_HOPS,)),
            pltpu.SemaphoreType.DMA((N_LAYERS * N_HOPS,)),
        ],
        compiler_params=pltpu.CompilerParams(collective_id=0),
    )(x, Win0, Wout0, Win1, Wout1, Win2, Wout2)


# device time: 39646 ns/iter; 1.9989x vs baseline; 1.9989x over previous
import jax
import jax.numpy as jnp
from jax import lax
from jax.experimental import pallas as pl
from jax.experimental.pallas import tpu as pltpu

N_DEV = 4
N_LAYERS = 3
N_SLOTS = N_LAYERS * 4


def kernel(x, Win0, Wout0, Win1, Wout1, Win2, Wout2):
    b, d = x.shape
    hb = b // 2

    def body(x_ref, win0, wout0, win1, wout1, win2, wout2, out_ref,
             sbuf, rbuf, send_sems, recv_sems):
        my = lax.axis_index("i")
        partner_a = my ^ 1
        partner_b = 3 - my

        barrier = pltpu.get_barrier_semaphore()
        for nbr in (partner_a, partner_b):
            pl.semaphore_signal(
                barrier, inc=1,
                device_id=(nbr,), device_id_type=pl.DeviceIdType.MESH,
            )
        pl.semaphore_wait(barrier, 2)

        def exchange(slot, value, partner):
            sbuf[slot, :, :] = value
            rdma = pltpu.make_async_remote_copy(
                src_ref=sbuf.at[slot],
                dst_ref=rbuf.at[slot],
                send_sem=send_sems.at[slot],
                recv_sem=recv_sems.at[slot],
                device_id=(partner,),
                device_id_type=pl.DeviceIdType.MESH,
            )
            rdma.start()
            return rdma

        xA = x_ref[:hb, :]
        xB = x_ref[hb:, :]
        layers = [(win0, wout0), (win1, wout1), (win2, wout2)]
        for l, (win, wout) in enumerate(layers):
            wv = win[:, :]
            wo = wout[:, :]
            s = l * 4

            pA = jnp.dot(
                jnp.maximum(jnp.dot(xA, wv, preferred_element_type=jnp.float32), 0.0),
                wo, preferred_element_type=jnp.float32)
            rA1 = exchange(s + 0, pA, partner_a)

            pB = jnp.dot(
                jnp.maximum(jnp.dot(xB, wv, preferred_element_type=jnp.float32), 0.0),
                wo, preferred_element_type=jnp.float32)
            rB1 = exchange(s + 1, pB, partner_b)

            rA1.wait()
            A2 = pA + rbuf[s + 0, :, :]
            rA2 = exchange(s + 2, A2, partner_b)

            rB1.wait()
            B2 = pB + rbuf[s + 1, :, :]
            rB2 = exchange(s + 3, B2, partner_a)

            rA2.wait()
            xA = A2 + rbuf[s + 2, :, :]
            rB2.wait()
            xB = B2 + rbuf[s + 3, :, :]

        out_ref[:hb, :] = xA
        out_ref[hb:, :] = xB

    return pl.pallas_call(
        body,
        out_shape=jax.ShapeDtypeStruct((b, d), jnp.float32),
        in_specs=[pl.BlockSpec(memory_space=pltpu.VMEM)] * 7,
        out_specs=pl.BlockSpec(memory_space=pltpu.VMEM),
        scratch_shapes=[
            pltpu.VMEM((N_SLOTS, hb, d), jnp.float32),
            pltpu.VMEM((N_SLOTS, hb, d), jnp.float32),
            pltpu.SemaphoreType.DMA((N_SLOTS,)),
            pltpu.SemaphoreType.DMA((N_SLOTS,)),
        ],
        compiler_params=pltpu.CompilerParams(collective_id=0),
    )(x, Win0, Wout0, Win1, Wout1, Win2, Wout2)


# device time: 33342 ns/iter; 2.3768x vs baseline; 1.1891x over previous
import jax
import jax.numpy as jnp
from jax import lax
from jax.experimental import pallas as pl
from jax.experimental.pallas import tpu as pltpu

N_DEV = 4
N_LAYERS = 3
N_CHUNKS = 4
N_SLOTS = N_LAYERS * N_CHUNKS * 2


def kernel(x, Win0, Wout0, Win1, Wout1, Win2, Wout2):
    b, d = x.shape
    cb = b // N_CHUNKS

    def body(x_ref, win0, wout0, win1, wout1, win2, wout2, out_ref,
             sbuf, rbuf, send_sems, recv_sems):
        my = lax.axis_index("i")
        partner_a = my ^ 1
        partner_b = 3 - my

        barrier = pltpu.get_barrier_semaphore()
        for nbr in (partner_a, partner_b):
            pl.semaphore_signal(
                barrier, inc=1,
                device_id=(nbr,), device_id_type=pl.DeviceIdType.MESH,
            )
        pl.semaphore_wait(barrier, 2)

        def exchange(slot, value, partner):
            sbuf[slot, :, :] = value
            rdma = pltpu.make_async_remote_copy(
                src_ref=sbuf.at[slot],
                dst_ref=rbuf.at[slot],
                send_sem=send_sems.at[slot],
                recv_sem=recv_sems.at[slot],
                device_id=(partner,),
                device_id_type=pl.DeviceIdType.MESH,
            )
            rdma.start()
            return rdma

        layers = [(win0, wout0), (win1, wout1), (win2, wout2)]
        p1 = [partner_a, partner_b, partner_a, partner_b]
        p2 = [partner_b, partner_a, partner_b, partner_a]

        def slot(l, c, ph):
            return (l * N_CHUNKS + c) * 2 + ph

        def fwd(l, xc):
            win, wout = layers[l]
            h = jnp.maximum(
                jnp.dot(xc, win[:, :], preferred_element_type=jnp.float32), 0.0)
            return jnp.dot(h, wout[:, :], preferred_element_type=jnp.float32)

        p = [None] * N_CHUNKS
        r1 = [None] * N_CHUNKS
        r2 = [None] * N_CHUNKS
        for c in range(N_CHUNKS):
            p[c] = fwd(0, x_ref[c * cb:(c + 1) * cb, :])
            r1[c] = exchange(slot(0, c, 0), p[c], p1[c])

        for l in range(N_LAYERS):
            for c in range(N_CHUNKS):
                r1[c].wait()
                p[c] = p[c] + rbuf[slot(l, c, 0), :, :]
                r2[c] = exchange(slot(l, c, 1), p[c], p2[c])
            for c in range(N_CHUNKS):
                r2[c].wait()
                xc = p[c] + rbuf[slot(l, c, 1), :, :]
                if l + 1 < N_LAYERS:
                    p[c] = fwd(l + 1, xc)
                    r1[c] = exchange(slot(l + 1, c, 0), p[c], p1[c])
                else:
                    out_ref[c * cb:(c + 1) * cb, :] = xc

    return pl.pallas_call(
        body,
        out_shape=jax.ShapeDtypeStruct((b, d), jnp.float32),
        in_specs=[pl.BlockSpec(memory_space=pltpu.VMEM)] * 7,
        out_specs=pl.BlockSpec(memory_space=pltpu.VMEM),
        scratch_shapes=[
            pltpu.VMEM((N_SLOTS, cb, d), jnp.float32),
            pltpu.VMEM((N_SLOTS, cb, d), jnp.float32),
            pltpu.SemaphoreType.DMA((N_SLOTS,)),
            pltpu.SemaphoreType.DMA((N_SLOTS,)),
        ],
        compiler_params=pltpu.CompilerParams(collective_id=0),
    )(x, Win0, Wout0, Win1, Wout1, Win2, Wout2)


# device time: 26750 ns/iter; 2.9625x vs baseline; 1.2464x over previous
import jax
import jax.numpy as jnp
from jax import lax
from jax.experimental import pallas as pl
from jax.experimental.pallas import tpu as pltpu

N_DEV = 4
N_LAYERS = 3
N_CHUNKS = 16
N_SLOTS = N_LAYERS * N_CHUNKS * 2


def kernel(x, Win0, Wout0, Win1, Wout1, Win2, Wout2):
    b, d = x.shape
    cb = b // N_CHUNKS

    def body(x_ref, win0, wout0, win1, wout1, win2, wout2, out_ref,
             sbuf, rbuf, send_sems, recv_sems):
        my = lax.axis_index("i")
        partner_a = my ^ 1
        partner_b = 3 - my

        barrier = pltpu.get_barrier_semaphore()
        for nbr in (partner_a, partner_b):
            pl.semaphore_signal(
                barrier, inc=1,
                device_id=(nbr,), device_id_type=pl.DeviceIdType.MESH,
            )
        pl.semaphore_wait(barrier, 2)

        def exchange(slot, value, partner):
            sbuf[slot, :, :] = value.astype(jnp.bfloat16)
            rdma = pltpu.make_async_remote_copy(
                src_ref=sbuf.at[slot],
                dst_ref=rbuf.at[slot],
                send_sem=send_sems.at[slot],
                recv_sem=recv_sems.at[slot],
                device_id=(partner,),
                device_id_type=pl.DeviceIdType.MESH,
            )
            rdma.start()
            return rdma

        layers = [(win0, wout0), (win1, wout1), (win2, wout2)]
        p1 = [partner_a if c % 2 == 0 else partner_b for c in range(N_CHUNKS)]
        p2 = [partner_b if c % 2 == 0 else partner_a for c in range(N_CHUNKS)]

        def slot(l, c, ph):
            return (l * N_CHUNKS + c) * 2 + ph

        def fwd(l, xc):
            win, wout = layers[l]
            h = jnp.maximum(
                jnp.dot(xc, win[:, :], preferred_element_type=jnp.float32), 0.0)
            return jnp.dot(h, wout[:, :], preferred_element_type=jnp.float32)

        GROUP = 4
        p = [None] * N_CHUNKS
        r1 = [None] * N_CHUNKS
        r2 = [None] * N_CHUNKS
        for g in range(N_CHUNKS // GROUP):
            c0 = g * GROUP
            pg = fwd(0, x_ref[c0 * cb:(c0 + GROUP) * cb, :])
            for j in range(GROUP):
                c = c0 + j
                p[c] = pg[j * cb:(j + 1) * cb, :]
                r1[c] = exchange(slot(0, c, 0), p[c], p1[c])

        for l in range(N_LAYERS):
            for c in range(N_CHUNKS):
                r1[c].wait()
                p[c] = p[c] + rbuf[slot(l, c, 0), :, :].astype(jnp.float32)
                r2[c] = exchange(slot(l, c, 1), p[c], p2[c])
            for g in range(N_CHUNKS // GROUP):
                c0 = g * GROUP
                xg = [None] * GROUP
                for j in range(GROUP):
                    c = c0 + j
                    r2[c].wait()
                    xg[j] = p[c] + rbuf[slot(l, c, 1), :, :].astype(jnp.float32)
                if l + 1 < N_LAYERS:
                    pg = fwd(l + 1, jnp.concatenate(xg, axis=0))
                    for j in range(GROUP):
                        c = c0 + j
                        p[c] = pg[j * cb:(j + 1) * cb, :]
                        r1[c] = exchange(slot(l + 1, c, 0), p[c], p1[c])
                else:
                    for j in range(GROUP):
                        c = c0 + j
                        out_ref[c * cb:(c + 1) * cb, :] = xg[j]

    return pl.pallas_call(
        body,
        out_shape=jax.ShapeDtypeStruct((b, d), jnp.float32),
        in_specs=[pl.BlockSpec(memory_space=pltpu.VMEM)] * 7,
        out_specs=pl.BlockSpec(memory_space=pltpu.VMEM),
        scratch_shapes=[
            pltpu.VMEM((N_SLOTS, cb, d), jnp.bfloat16),
            pltpu.VMEM((N_SLOTS, cb, d), jnp.bfloat16),
            pltpu.SemaphoreType.DMA((N_SLOTS,)),
            pltpu.SemaphoreType.DMA((N_SLOTS,)),
        ],
        compiler_params=pltpu.CompilerParams(collective_id=0),
    )(x, Win0, Wout0, Win1, Wout1, Win2, Wout2)
